# baseline (speedup 1.0000x reference)
import functools

import jax
import jax.numpy as jnp
from jax.experimental import pallas as pl
from jax.experimental.pallas import tpu as pltpu


def _stats(y, tcol):
    cols = jax.lax.broadcasted_iota(jnp.int32, y.shape, 1)
    eqm = cols == tcol
    L1 = jnp.sum(jnp.where(eqm, y, 0.0), axis=1, keepdims=True)
    M0 = jnp.max(y, axis=1, keepdims=True)
    M1 = jnp.max(jnp.where(eqm, -jnp.inf, y), axis=1, keepdims=True)
    lse = jnp.log(jnp.sum(jnp.exp(y - M0), axis=1, keepdims=True)) + M0
    f1 = L1 == M0
    h = jnp.maximum(1.0 - L1 + jnp.where(f1, M1, M0), 0.0)
    s = jnp.maximum(1.0 - L1 + jnp.where(f1, M1, lse), 0.0)
    nwrong = jnp.sum(jnp.where(f1, 0.0, 1.0))
    return h, s, nwrong


def _fused_body(N, H, ta_ref, ya_ref, tb_ref, yb_ref, tc_ref, yc_ref,
                td_ref, yd_ref, out_ref, h_scr, s_scr, e_scr):
    i = pl.program_id(0)

    @pl.when(i == 0)
    def _init():
        e_scr[0] = 0.0

    BR = ya_ref.shape[0]
    r16 = BR // 128
    off = h_scr.shape[0] // 4
    esum = e_scr[0]
    for q, (t_ref, y_ref) in enumerate(
            [(ta_ref, ya_ref), (tb_ref, yb_ref),
             (tc_ref, yc_ref), (td_ref, yd_ref)]):
        hq, sq, wq = _stats(y_ref[...], t_ref[...])
        esum += wq
        h_scr[pl.ds(q * off + i * r16, r16), :] = hq.reshape(r16, 128)
        s_scr[pl.ds(q * off + i * r16, r16), :] = sq.reshape(r16, 128)
    e_scr[0] = esum

    @pl.when(i == H - 1)
    def _select():
        h = h_scr[...]
        s = s_scr[...]
        R = h.shape[0]
        C = jnp.float32(N) + e_scr[0]
        bits = jax.lax.bitcast_convert_type(h, jnp.int32)
        aidx = jax.lax.broadcasted_iota(jnp.int32, (R, 128), 0)
        bidx = jax.lax.broadcasted_iota(jnp.int32, (R, 128), 1)
        idx = aidx * 128 + bidx

        def cnt_lt(v):
            return jnp.sum(jnp.where(bits < v, 1.0, 0.0))

        def sum_h_lt(v):
            return jnp.sum(jnp.where(bits < v, h, 0.0))

        hp1 = h + 1.0
        Cp1 = C + 1.0

        def feas(v):
            return jnp.sum(jnp.where(bits < v, hp1, 0.0)) <= Cp1

        def ph1(b, v):
            p = 28 - 2 * b
            u = jnp.left_shift(jnp.int32(1), p)
            f1_ = feas(v + u)
            f2_ = feas(v + 2 * u)
            f3_ = feas(v + 3 * u)
            inc = jnp.where(f3_, 3, jnp.where(f2_, 2, jnp.where(f1_, 1, 0)))
            return v + inc.astype(jnp.int32) * u

        v30 = jnp.left_shift(jnp.int32(1), 30)
        vstar = jnp.where(feas(v30), v30, jnp.int32(0))
        vstar = jax.lax.fori_loop(0, 15, ph1, vstar)
        hval = jax.lax.bitcast_convert_type(vstar, jnp.float32)
        n_lt = cnt_lt(vstar)
        s_lt = sum_h_lt(vstar)
        cnt_tie = jnp.sum(jnp.where(bits == vstar, 1.0, 0.0))
        m = jnp.floor((C + 1.0 - n_lt - s_lt) / (hval + 1.0))
        m = jnp.clip(m, 0.0, cnt_tie)
        kstar = n_lt + m
        Sstar = s_lt + m * hval
        total = jnp.sum(h)
        upb = jnp.where(kstar == 0.0, total <= C, Sstar <= C - kstar)
        kf = jnp.minimum(kstar + jnp.where(upb, 1.0, 0.0), jnp.float32(N))
        need = m + (kf - kstar)
        over = need > cnt_tie
        nxt = jnp.min(jnp.where(bits > vstar, bits, jnp.int32(2**31 - 1)))
        w = jnp.where(over, nxt, vstar)
        m2 = jnp.where(over, 1.0, need)
        sum_s_lt = jnp.sum(jnp.where(bits < w, s, 0.0))
        tie = bits == w

        def tcnt(q):
            return jnp.sum(jnp.where(tie & (idx < q), 1.0, 0.0)) < m2

        def ph3(b, q):
            p = 12 - 2 * b
            u = jnp.left_shift(jnp.int32(1), p)
            g1 = tcnt(q + u)
            g2 = tcnt(q + 2 * u)
            g3 = tcnt(q + 3 * u)
            inc = jnp.where(g3, 3, jnp.where(g2, 2, jnp.where(g1, 1, 0)))
            return q + inc.astype(jnp.int32) * u

        q14 = jnp.left_shift(jnp.int32(1), 14)
        qstar = jnp.where(tcnt(q14), q14, jnp.int32(0))
        qstar = jax.lax.fori_loop(0, 7, ph3, qstar)
        sum_s_tie = jnp.sum(jnp.where(tie & (idx <= qstar), s, 0.0))
        res = (sum_s_lt + sum_s_tie) / kf
        out_ref[...] = jnp.full((1, 1), res, dtype=jnp.float32)


def _impl(y_1, t, interpret=False):
    N, K = y_1.shape
    BR = 512
    H = N // BR // 4
    t2 = t.reshape(N, 1)
    specs = []
    for q in range(4):
        specs.append(pl.BlockSpec((BR, 1), lambda i, q=q, H=H: (i + q * H, 0)))
        specs.append(pl.BlockSpec((BR, K), lambda i, q=q, H=H: (i + q * H, 0)))
    out = pl.pallas_call(
        functools.partial(_fused_body, N, H),
        grid=(H,),
        in_specs=specs,
        out_specs=pl.BlockSpec((1, 1), lambda i: (0, 0)),
        out_shape=jax.ShapeDtypeStruct((1, 1), jnp.float32),
        scratch_shapes=[
            pltpu.VMEM((N // 128, 128), jnp.float32),
            pltpu.VMEM((N // 128, 128), jnp.float32),
            pltpu.SMEM((1,), jnp.float32),
        ],
        interpret=interpret,
    )(t2, y_1, t2, y_1, t2, y_1, t2, y_1)
    return out[0, 0]


def kernel(y_1, t):
    return _impl(y_1, t)


# speedup vs baseline: 1.0112x; 1.0112x over previous
import functools

import jax
import jax.numpy as jnp
from jax.experimental import pallas as pl
from jax.experimental.pallas import tpu as pltpu


def _stats(y, tcol):
    cols = jax.lax.broadcasted_iota(jnp.int32, y.shape, 1)
    eqm = cols == tcol
    L1 = jnp.sum(jnp.where(eqm, y, 0.0), axis=1, keepdims=True)
    M0 = jnp.max(y, axis=1, keepdims=True)
    M1 = jnp.max(jnp.where(eqm, -jnp.inf, y), axis=1, keepdims=True)
    lse = jnp.log(jnp.sum(jnp.exp(y - M0), axis=1, keepdims=True)) + M0
    f1 = L1 == M0
    h = jnp.maximum(1.0 - L1 + jnp.where(f1, M1, M0), 0.0)
    s = jnp.maximum(1.0 - L1 + jnp.where(f1, M1, lse), 0.0)
    nwrong = jnp.sum(jnp.where(f1, 0.0, 1.0))
    return h, s, nwrong


def _fused_body(N, H, ta_ref, ya_ref, tb_ref, yb_ref, out_ref,
                h_scr, s_scr, e_scr):
    i = pl.program_id(0)

    @pl.when(i == 0)
    def _init():
        e_scr[0] = 0.0

    ha, sa, wa = _stats(ya_ref[...], ta_ref[...])
    hb, sb, wb = _stats(yb_ref[...], tb_ref[...])
    e_scr[0] += wa + wb
    BR = ya_ref.shape[0]
    r16 = BR // 128
    off = h_scr.shape[0] // 2
    h_scr[pl.ds(i * r16, r16), :] = ha.reshape(r16, 128)
    s_scr[pl.ds(i * r16, r16), :] = sa.reshape(r16, 128)
    h_scr[pl.ds(off + i * r16, r16), :] = hb.reshape(r16, 128)
    s_scr[pl.ds(off + i * r16, r16), :] = sb.reshape(r16, 128)

    @pl.when(i == H - 1)
    def _select():
        h = h_scr[...]
        s = s_scr[...]
        R = h.shape[0]
        C = jnp.float32(N) + e_scr[0]
        bits = jax.lax.bitcast_convert_type(h, jnp.int32)
        aidx = jax.lax.broadcasted_iota(jnp.int32, (R, 128), 0)
        bidx = jax.lax.broadcasted_iota(jnp.int32, (R, 128), 1)
        idx = aidx * 128 + bidx

        def cnt_lt(v):
            return jnp.sum(jnp.where(bits < v, 1.0, 0.0))

        def sum_h_lt(v):
            return jnp.sum(jnp.where(bits < v, h, 0.0))

        hp1 = h + 1.0
        Cp1 = C + 1.0

        def feas(v):
            return jnp.sum(jnp.where(bits < v, hp1, 0.0)) <= Cp1

        def ph1(b, v):
            p = 28 - 2 * b
            u = jnp.left_shift(jnp.int32(1), p)
            f1_ = feas(v + u)
            f2_ = feas(v + 2 * u)
            f3_ = feas(v + 3 * u)
            inc = jnp.where(f3_, 3, jnp.where(f2_, 2, jnp.where(f1_, 1, 0)))
            return v + inc.astype(jnp.int32) * u

        v30 = jnp.left_shift(jnp.int32(1), 30)
        vstar = jnp.where(feas(v30), v30, jnp.int32(0))
        vstar = jax.lax.fori_loop(0, 15, ph1, vstar)
        hval = jax.lax.bitcast_convert_type(vstar, jnp.float32)
        n_lt = cnt_lt(vstar)
        s_lt = sum_h_lt(vstar)
        cnt_tie = jnp.sum(jnp.where(bits == vstar, 1.0, 0.0))
        m = jnp.floor((C + 1.0 - n_lt - s_lt) / (hval + 1.0))
        m = jnp.clip(m, 0.0, cnt_tie)
        kstar = n_lt + m
        Sstar = s_lt + m * hval
        total = jnp.sum(h)
        upb = jnp.where(kstar == 0.0, total <= C, Sstar <= C - kstar)
        kf = jnp.minimum(kstar + jnp.where(upb, 1.0, 0.0), jnp.float32(N))
        need = m + (kf - kstar)
        over = need > cnt_tie
        nxt = jnp.min(jnp.where(bits > vstar, bits, jnp.int32(2**31 - 1)))
        w = jnp.where(over, nxt, vstar)
        m2 = jnp.where(over, 1.0, need)
        sum_s_lt = jnp.sum(jnp.where(bits < w, s, 0.0))
        tie = bits == w

        def tcnt(q):
            return jnp.sum(jnp.where(tie & (idx < q), 1.0, 0.0)) < m2

        def ph3(b, q):
            p = 12 - 2 * b
            u = jnp.left_shift(jnp.int32(1), p)
            g1 = tcnt(q + u)
            g2 = tcnt(q + 2 * u)
            g3 = tcnt(q + 3 * u)
            inc = jnp.where(g3, 3, jnp.where(g2, 2, jnp.where(g1, 1, 0)))
            return q + inc.astype(jnp.int32) * u

        q14 = jnp.left_shift(jnp.int32(1), 14)
        qstar = jnp.where(tcnt(q14), q14, jnp.int32(0))
        qstar = jax.lax.fori_loop(0, 7, ph3, qstar)
        sum_s_tie = jnp.sum(jnp.where(tie & (idx <= qstar), s, 0.0))
        res = (sum_s_lt + sum_s_tie) / kf
        out_ref[...] = jnp.full((1, 1), res, dtype=jnp.float32)


def _impl(y_1, t):
    N, K = y_1.shape
    BR = 1024
    H = N // BR // 2
    t2 = t.reshape(N, 1)
    out = pl.pallas_call(
        functools.partial(_fused_body, N, H),
        grid=(H,),
        in_specs=[
            pl.BlockSpec((BR, 1), lambda i: (i, 0)),
            pl.BlockSpec((BR, K), lambda i: (i, 0)),
            pl.BlockSpec((BR, 1), lambda i, H=H: (i + H, 0)),
            pl.BlockSpec((BR, K), lambda i, H=H: (i + H, 0)),
        ],
        out_specs=pl.BlockSpec((1, 1), lambda i: (0, 0)),
        out_shape=jax.ShapeDtypeStruct((1, 1), jnp.float32),
        scratch_shapes=[
            pltpu.VMEM((N // 128, 128), jnp.float32),
            pltpu.VMEM((N // 128, 128), jnp.float32),
            pltpu.SMEM((1,), jnp.float32),
        ],
    )(t2, y_1, t2, y_1)
    return out[0, 0]


def kernel(y_1, t):
    return _impl(y_1, t)
